# baseline (speedup 1.0000x reference)
import functools

import jax
import jax.numpy as jnp
from jax.experimental import pallas as pl
from jax.experimental.pallas import tpu as pltpu

_N = 10000
_E = 320000
_LG = 640000
_D = 128
_DE = 16
_NIT = 4
_EPS = 1e-5

_BE = 2560
_NBLK = _E // _BE


def _node_mm_body(x_ref, wi_ref, wj_ref, ai_ref, aj_ref):
    xv = x_ref[...]
    ai_ref[...] = jnp.dot(xv, wi_ref[...], preferred_element_type=jnp.float32)
    aj_ref[...] = jnp.dot(xv, wj_ref[...], preferred_element_type=jnp.float32)


def _node_mm(x, w_i, w_j):
    return pl.pallas_call(
        _node_mm_body,
        out_shape=(jax.ShapeDtypeStruct((_N, _D), jnp.float32),
                   jax.ShapeDtypeStruct((_N, _D), jnp.float32)),
    )(x, w_i, w_j)


def _edge_alpha_body(g_ref, ef_ref, wsml_ref, we_ref, bias_ref, bsml_ref,
                     be_ref, p_ref, out_ref):
    g = g_ref[...] + bias_ref[...]
    a = jnp.where(g >= 0, g, p_ref[...] * g)
    m = jnp.dot(a, wsml_ref[...], preferred_element_type=jnp.float32) + bsml_ref[...]
    ef = jnp.dot(ef_ref[...], we_ref[...], preferred_element_type=jnp.float32) + be_ref[...]
    s = jnp.sum(m * ef, axis=-1)
    out_ref[...] = s.reshape(1, 1, _BE)


def _edge_alpha(g, edge_feats, W_sml, W_e, bias2, bsml2, be2, p2):
    out = pl.pallas_call(
        _edge_alpha_body,
        grid=(_NBLK,),
        in_specs=[
            pl.BlockSpec((_BE, _D), lambda i: (i, 0)),
            pl.BlockSpec((_BE, _DE), lambda i: (i, 0)),
            pl.BlockSpec((_D, _D), lambda i: (0, 0)),
            pl.BlockSpec((_DE, _D), lambda i: (0, 0)),
            pl.BlockSpec((1, _D), lambda i: (0, 0)),
            pl.BlockSpec((1, _D), lambda i: (0, 0)),
            pl.BlockSpec((1, _D), lambda i: (0, 0)),
            pl.BlockSpec((1, _D), lambda i: (0, 0)),
        ],
        out_specs=pl.BlockSpec((1, 1, _BE), lambda i: (i, 0, 0)),
        out_shape=jax.ShapeDtypeStruct((_NBLK, 1, _BE), jnp.float32),
    )(g, edge_feats, W_sml, W_e, bias2, bsml2, be2, p2)
    return out.reshape(_E)


def _bn_in(xv, g, b):
    mean = jnp.mean(xv, axis=0, keepdims=True)
    var = jnp.mean((xv - mean) ** 2, axis=0, keepdims=True)
    return g * (xv - mean) / jnp.sqrt(var + _EPS) + b


def _mlp_body(xn_ref, bn1g, bn1b, w1, b1, bn2g, bn2b, p2, w2, b2,
              bn3g, bn3b, p3, w3, b3, bn4g, bn4b, p4, w4, b4, out_ref):
    xn = xn_ref[...]
    h = jnp.dot(_bn_in(xn, bn1g[...], bn1b[...]), w1[...],
                preferred_element_type=jnp.float32) + b1[...]
    t = _bn_in(h, bn2g[...], bn2b[...])
    t = jnp.where(t >= 0, t, p2[...] * t)
    h2 = jnp.dot(t, w2[...], preferred_element_type=jnp.float32) + b2[...]
    t = _bn_in(h2, bn3g[...], bn3b[...])
    t = jnp.where(t >= 0, t, p3[...] * t)
    h3 = jnp.dot(t, w3[...], preferred_element_type=jnp.float32) + b3[...]
    h = (h3 + h) / 2
    t = _bn_in(h, bn4g[...], bn4b[...])
    t = jnp.where(t >= 0, t, p4[...] * t)
    h4 = jnp.dot(t, w4[...], preferred_element_type=jnp.float32) + b4[...]
    out_ref[...] = (h4 + h) / 2


def _mlp(xn, *params):
    return pl.pallas_call(
        _mlp_body,
        out_shape=jax.ShapeDtypeStruct((_N, _D), jnp.float32),
    )(xn, *params)


def _row(v):
    return jnp.broadcast_to(v.reshape(1, -1), (1, _D)) if v.shape[0] != _D \
        else v.reshape(1, _D)


def kernel(x, edge_index, edge_feats, line_graph_edge_index, xchemfea,
           w_i, w_j, bias, W_e, b_e, p_sml, W_sml, b_sml,
           bn1_g, bn1_b, W1, b1,
           bn2_g, bn2_b, p2, W2, b2,
           bn3_g, bn3_b, p3, W3, b3,
           bn4_g, bn4_b, p4, W4, b4):
    src = edge_index[0]
    dst = edge_index[1]

    ai, aj = _node_mm(x, w_i, w_j)
    g = ai[dst] + aj[src]
    alpha = _edge_alpha(g, edge_feats, W_sml, W_e,
                        _row(bias), _row(b_sml), _row(b_e), _row(p_sml))

    deg = jnp.zeros((_N,), jnp.float32).at[dst].add(1.0)
    alpha = alpha / deg[src]
    ew = jax.nn.sigmoid(alpha)

    edge_attr = x[src] * ew[:, None]
    out = edge_attr
    lg_src = line_graph_edge_index[0]
    lg_dst = line_graph_edge_index[1]
    for _ in range(_NIT):
        agg = jnp.zeros_like(edge_attr).at[lg_dst].add(out[lg_src])
        out = edge_attr + agg * ew[:, None]
    xn = x + jnp.zeros_like(x).at[dst].add(out)

    return _mlp(xn, _row(bn1_g), _row(bn1_b), W1, _row(b1),
                _row(bn2_g), _row(bn2_b), _row(p2), W2, _row(b2),
                _row(bn3_g), _row(bn3_b), _row(p3), W3, _row(b3),
                _row(bn4_g), _row(bn4_b), _row(p4), W4, _row(b4))
